# Initial kernel scaffold; baseline (speedup 1.0000x reference)
import functools

import jax
import jax.numpy as jnp
from jax import lax
from jax.experimental import pallas as pl
from jax.experimental.pallas import tpu as pltpu
from jax.experimental.pallas import tpu_sc as plsc

N = 10000
E = 320000
IN_DIM = 128
HID = 64
ALIGN = 32
NUM_CLASSES = 7

NC = 2
NS = 16
CH = 128
NCH = 80
E_PAD = NC * NS * NCH * CH
DUMP = N
SROWS = 10240
RPT = SROWS // NS

BR = 2000
GRID = N // BR

_mesh = plsc.VectorSubcoreMesh(core_axis_name="c", subcore_axis_name="s")


@functools.partial(
    pl.kernel,
    out_type=jax.ShapeDtypeStruct((NC, SROWS, 16), jnp.float32),
    mesh=_mesh,
    scratch_types=[
        pltpu.VMEM((NCH, CH), jnp.int32),
        pltpu.VMEM((CH, 16), jnp.float32),
    ],
)
def _deg_kernel(dst_hbm, zeros_hbm, ones_hbm, deg_out, dst_v, ones_v):
    cid = lax.axis_index("c")
    sid = lax.axis_index("s")
    pltpu.sync_copy(dst_hbm.at[cid, sid], dst_v)
    pltpu.sync_copy(ones_hbm, ones_v)

    def acc_body(acc_sh):
        pltpu.sync_copy(zeros_hbm, acc_sh.at[pl.ds(sid * RPT, RPT)])
        plsc.subcore_barrier()

        def chunk(j, _):
            pltpu.sync_copy(ones_v, acc_sh.at[dst_v.at[j]], add=True)
            return ()

        lax.fori_loop(0, NCH, chunk, ())
        plsc.subcore_barrier()
        pltpu.sync_copy(acc_sh.at[pl.ds(sid * RPT, RPT)],
                        deg_out.at[cid, pl.ds(sid * RPT, RPT)])

    pl.run_scoped(acc_body, pltpu.VMEM_SHARED((SROWS, 16), jnp.float32))


@functools.partial(
    pl.kernel,
    out_type=jax.ShapeDtypeStruct((NC, SROWS, HID), jnp.float32),
    mesh=_mesh,
    scratch_types=[
        pltpu.VMEM((NCH, CH), jnp.int32),
        pltpu.VMEM((NCH, CH), jnp.int32),
        pltpu.VMEM((CH, HID), jnp.float32),
        pltpu.VMEM((CH, HID), jnp.float32),
        pltpu.SemaphoreType.DMA,
        pltpu.SemaphoreType.DMA,
    ],
)
def _msg_kernel(src_hbm, dst_hbm, y_hbm, zeros_hbm, s_out,
                src_v, dst_v, buf0, buf1, sem0, sem1):
    cid = lax.axis_index("c")
    sid = lax.axis_index("s")
    pltpu.sync_copy(src_hbm.at[cid, sid], src_v)
    pltpu.sync_copy(dst_hbm.at[cid, sid], dst_v)

    def acc_body(acc_sh):
        pltpu.sync_copy(zeros_hbm, acc_sh.at[pl.ds(sid * RPT, RPT)])
        plsc.subcore_barrier()

        pltpu.async_copy(y_hbm.at[src_v.at[0]], buf0, sem0)
        pltpu.async_copy(y_hbm.at[src_v.at[1]], buf1, sem1)

        def pair(i, _):
            j0 = i * 2
            for b, (buf, sem) in enumerate(((buf0, sem0), (buf1, sem1))):
                j = j0 + b
                pltpu.make_async_copy(y_hbm.at[src_v.at[j]], buf, sem).wait()
                pltpu.sync_copy(buf, acc_sh.at[dst_v.at[j]], add=True)

                @pl.when(j + 2 < NCH)
                def _():
                    pltpu.async_copy(y_hbm.at[src_v.at[j + 2]], buf, sem)

            return ()

        lax.fori_loop(0, NCH // 2, pair, ())
        plsc.subcore_barrier()
        pltpu.sync_copy(acc_sh.at[pl.ds(sid * RPT, RPT)],
                        s_out.at[cid, pl.ds(sid * RPT, RPT)])

    pl.run_scoped(acc_body, pltpu.VMEM_SHARED((SROWS, HID), jnp.float32))


def _proj_body(x_ref, wg_ref, wps_ref, bps_ref, xw_ref, zsem_ref):
    x = x_ref[...]
    xw_ref[...] = jnp.dot(x, wg_ref[...], preferred_element_type=jnp.float32)
    zsem_ref[...] = (
        jnp.dot(x, wps_ref[...], preferred_element_type=jnp.float32)
        + bps_ref[...]
    )


def _proj(x, W_gcn, W_ps, b_ps):
    return pl.pallas_call(
        _proj_body,
        grid=(GRID,),
        in_specs=[
            pl.BlockSpec((BR, IN_DIM), lambda i: (i, 0)),
            pl.BlockSpec((IN_DIM, HID), lambda i: (0, 0)),
            pl.BlockSpec((IN_DIM, ALIGN), lambda i: (0, 0)),
            pl.BlockSpec((1, ALIGN), lambda i: (0, 0)),
        ],
        out_specs=[
            pl.BlockSpec((BR, HID), lambda i: (i, 0)),
            pl.BlockSpec((BR, ALIGN), lambda i: (i, 0)),
        ],
        out_shape=[
            jax.ShapeDtypeStruct((N, HID), jnp.float32),
            jax.ShapeDtypeStruct((N, ALIGN), jnp.float32),
        ],
    )(x, W_gcn, W_ps, b_ps)


def _scale_body(dp_ref, xw_ref, y_ref, dinv_ref):
    deg = dp_ref[0, :, 0:1] + dp_ref[1, :, 0:1] + 1.0
    dinv = lax.rsqrt(deg)
    y_ref[...] = dinv * xw_ref[...]
    dinv_ref[...] = jnp.broadcast_to(dinv, dinv_ref.shape)


def _scale(deg_parts, xw):
    return pl.pallas_call(
        _scale_body,
        grid=(GRID,),
        in_specs=[
            pl.BlockSpec((2, BR, 16), lambda i: (0, i, 0)),
            pl.BlockSpec((BR, HID), lambda i: (i, 0)),
        ],
        out_specs=[
            pl.BlockSpec((BR, HID), lambda i: (i, 0)),
            pl.BlockSpec((BR, 16), lambda i: (i, 0)),
        ],
        out_shape=[
            jax.ShapeDtypeStruct((N, HID), jnp.float32),
            jax.ShapeDtypeStruct((N, 16), jnp.float32),
        ],
    )(deg_parts, xw)


def _head_body(s_ref, y_ref, dinv_ref, zsem_ref, bg_ref, wpt_ref, bpt_ref,
               wcls_ref, bcls_ref, logits_ref, an_ref, ztopo_ref):
    dinv = dinv_ref[:, 0:1]
    s_tot = s_ref[0] + s_ref[1] + y_ref[...]
    h = jnp.maximum(dinv * s_tot + bg_ref[...], 0.0)
    z_topo = (
        jnp.dot(h, wpt_ref[...], preferred_element_type=jnp.float32)
        + bpt_ref[...]
    )
    logits_ref[...] = (
        jnp.dot(z_topo, wcls_ref[...], preferred_element_type=jnp.float32)
        + bcls_ref[...]
    )
    diff = z_topo - zsem_ref[...]
    an_ref[...] = jnp.sqrt(jnp.sum(diff * diff, axis=1))
    ztopo_ref[...] = z_topo


def _heads(s_parts, y, dinv, z_sem, b_gcn, W_pt, b_pt, W_cls, b_cls):
    return pl.pallas_call(
        _head_body,
        grid=(GRID,),
        in_specs=[
            pl.BlockSpec((2, BR, HID), lambda i: (0, i, 0)),
            pl.BlockSpec((BR, HID), lambda i: (i, 0)),
            pl.BlockSpec((BR, 16), lambda i: (i, 0)),
            pl.BlockSpec((BR, ALIGN), lambda i: (i, 0)),
            pl.BlockSpec((1, HID), lambda i: (0, 0)),
            pl.BlockSpec((HID, ALIGN), lambda i: (0, 0)),
            pl.BlockSpec((1, ALIGN), lambda i: (0, 0)),
            pl.BlockSpec((ALIGN, NUM_CLASSES), lambda i: (0, 0)),
            pl.BlockSpec((1, NUM_CLASSES), lambda i: (0, 0)),
        ],
        out_specs=[
            pl.BlockSpec((BR, NUM_CLASSES), lambda i: (i, 0)),
            pl.BlockSpec((BR,), lambda i: (i,)),
            pl.BlockSpec((BR, ALIGN), lambda i: (i, 0)),
        ],
        out_shape=[
            jax.ShapeDtypeStruct((N, NUM_CLASSES), jnp.float32),
            jax.ShapeDtypeStruct((N,), jnp.float32),
            jax.ShapeDtypeStruct((N, ALIGN), jnp.float32),
        ],
    )(s_parts, y, dinv, z_sem, b_gcn, W_pt, b_pt, W_cls, b_cls)


def kernel(x, edge_index, W_gcn, b_gcn, W_pt, b_pt, W_ps, b_ps, W_cls, b_cls):
    pad = E_PAD - E
    src_r = jnp.concatenate(
        [edge_index[0], jnp.zeros((pad,), jnp.int32)]).reshape(NC, NS, NCH, CH)
    dst_r = jnp.concatenate(
        [edge_index[1], jnp.full((pad,), DUMP, jnp.int32)]).reshape(
            NC, NS, NCH, CH)

    zeros16 = jnp.zeros((RPT, 16), jnp.float32)
    zeros64 = jnp.zeros((RPT, HID), jnp.float32)
    ones16 = jnp.ones((CH, 16), jnp.float32)

    deg_parts = _deg_kernel(dst_r, zeros16, ones16)
    xw, z_sem = _proj(x, W_gcn, W_ps, b_ps.reshape(1, ALIGN))
    y, dinv = _scale(deg_parts, xw)
    s_parts = _msg_kernel(src_r, dst_r, y, zeros64)
    logits, anomaly, z_topo = _heads(
        s_parts, y, dinv, z_sem, b_gcn.reshape(1, HID), W_pt,
        b_pt.reshape(1, ALIGN), W_cls, b_cls.reshape(1, NUM_CLASSES))
    return (logits, anomaly, z_topo, z_sem)


# speedup vs baseline: 23.7975x; 23.7975x over previous
import functools

import jax
import jax.numpy as jnp
from jax import lax
from jax.experimental import pallas as pl
from jax.experimental.pallas import tpu as pltpu
from jax.experimental.pallas import tpu_sc as plsc

N = 10000
E = 320000
IN_DIM = 128
HID = 64
ALIGN = 32
NUM_CLASSES = 7

NC = 2
NS = 16
CH = 128
NCH = 80
E_PAD = NC * NS * NCH * CH
DUMP = N
SROWS = 10240
RPT = SROWS // NS

BR = 2048
GRID = (N + BR - 1) // BR

_mesh = plsc.VectorSubcoreMesh(core_axis_name="c", subcore_axis_name="s")
_sc_params = pltpu.CompilerParams(use_tc_tiling_on_sc=False)


@functools.partial(
    pl.kernel,
    out_type=jax.ShapeDtypeStruct((NC, SROWS, 16), jnp.float32),
    mesh=_mesh,
    scratch_types=[
        pltpu.VMEM((NCH, CH), jnp.int32),
        pltpu.VMEM((CH, 16), jnp.float32),
        pltpu.VMEM_SHARED((SROWS, 16), jnp.float32),
    ],
    compiler_params=_sc_params,
)
def _deg_kernel(dst_hbm, zeros_hbm, ones_hbm, deg_out, dst_v, ones_v, acc_sh):
    cid = lax.axis_index("c")
    sid = lax.axis_index("s")
    pltpu.sync_copy(dst_hbm.at[cid, sid], dst_v)
    pltpu.sync_copy(ones_hbm, ones_v)

    pltpu.sync_copy(zeros_hbm, acc_sh.at[pl.ds(sid * RPT, RPT)])
    plsc.subcore_barrier()

    def chunk(j, _):
        pltpu.sync_copy(ones_v, acc_sh.at[dst_v.at[j]], add=True)
        return ()

    lax.fori_loop(0, NCH, chunk, ())
    plsc.subcore_barrier()
    pltpu.sync_copy(acc_sh.at[pl.ds(sid * RPT, RPT)],
                    deg_out.at[cid, pl.ds(sid * RPT, RPT)])


@functools.partial(
    pl.kernel,
    out_type=jax.ShapeDtypeStruct((NC, SROWS, HID), jnp.float32),
    mesh=_mesh,
    scratch_types=[
        pltpu.VMEM((NCH, CH), jnp.int32),
        pltpu.VMEM((NCH, CH), jnp.int32),
        pltpu.VMEM((CH, HID), jnp.float32),
        pltpu.VMEM((CH, HID), jnp.float32),
        pltpu.SemaphoreType.DMA,
        pltpu.SemaphoreType.DMA,
        pltpu.VMEM_SHARED((SROWS, HID), jnp.float32),
    ],
    compiler_params=_sc_params,
)
def _msg_kernel(src_hbm, dst_hbm, y_hbm, zeros_hbm, s_out,
                src_v, dst_v, buf0, buf1, sem0, sem1, acc_sh):
    cid = lax.axis_index("c")
    sid = lax.axis_index("s")
    pltpu.sync_copy(src_hbm.at[cid, sid], src_v)
    pltpu.sync_copy(dst_hbm.at[cid, sid], dst_v)

    pltpu.sync_copy(zeros_hbm, acc_sh.at[pl.ds(sid * RPT, RPT)])
    plsc.subcore_barrier()

    pltpu.async_copy(y_hbm.at[src_v.at[0]], buf0, sem0)
    pltpu.async_copy(y_hbm.at[src_v.at[1]], buf1, sem1)

    def pair(i, _):
        j0 = i * 2
        for b, (buf, sem) in enumerate(((buf0, sem0), (buf1, sem1))):
            j = j0 + b
            pltpu.make_async_copy(y_hbm.at[src_v.at[j]], buf, sem).wait()
            pltpu.sync_copy(buf, acc_sh.at[dst_v.at[j]], add=True)

            @pl.when(j + 2 < NCH)
            def _():
                pltpu.async_copy(y_hbm.at[src_v.at[j + 2]], buf, sem)

        return ()

    lax.fori_loop(0, NCH // 2, pair, ())
    plsc.subcore_barrier()
    pltpu.sync_copy(acc_sh.at[pl.ds(sid * RPT, RPT)],
                    s_out.at[cid, pl.ds(sid * RPT, RPT)])


def _proj_body(x_ref, wg_ref, wps_ref, bps_ref, xw_ref, zsem_ref):
    x = x_ref[...]
    xw_ref[...] = jnp.dot(x, wg_ref[...], preferred_element_type=jnp.float32)
    zsem_ref[...] = (
        jnp.dot(x, wps_ref[...], preferred_element_type=jnp.float32)
        + bps_ref[...]
    )


def _proj(x, W_gcn, W_ps, b_ps):
    return pl.pallas_call(
        _proj_body,
        grid=(GRID,),
        in_specs=[
            pl.BlockSpec((BR, IN_DIM), lambda i: (i, 0)),
            pl.BlockSpec((IN_DIM, HID), lambda i: (0, 0)),
            pl.BlockSpec((IN_DIM, ALIGN), lambda i: (0, 0)),
            pl.BlockSpec((1, ALIGN), lambda i: (0, 0)),
        ],
        out_specs=[
            pl.BlockSpec((BR, HID), lambda i: (i, 0)),
            pl.BlockSpec((BR, ALIGN), lambda i: (i, 0)),
        ],
        out_shape=[
            jax.ShapeDtypeStruct((N, HID), jnp.float32),
            jax.ShapeDtypeStruct((N, ALIGN), jnp.float32),
        ],
    )(x, W_gcn, W_ps, b_ps)


def _scale_body(dp_ref, xw_ref, y_ref, dinv_ref):
    deg = dp_ref[0, :, 0:1] + dp_ref[1, :, 0:1] + 1.0
    dinv = lax.rsqrt(deg)
    y_ref[...] = dinv * xw_ref[...]
    dinv_ref[...] = jnp.broadcast_to(dinv, dinv_ref.shape)


def _scale(deg_parts, xw):
    return pl.pallas_call(
        _scale_body,
        grid=(GRID,),
        in_specs=[
            pl.BlockSpec((2, BR, 16), lambda i: (0, i, 0)),
            pl.BlockSpec((BR, HID), lambda i: (i, 0)),
        ],
        out_specs=[
            pl.BlockSpec((BR, HID), lambda i: (i, 0)),
            pl.BlockSpec((BR, 16), lambda i: (i, 0)),
        ],
        out_shape=[
            jax.ShapeDtypeStruct((N, HID), jnp.float32),
            jax.ShapeDtypeStruct((N, 16), jnp.float32),
        ],
    )(deg_parts, xw)


def _head_body(s_ref, y_ref, dinv_ref, zsem_ref, bg_ref, wpt_ref, bpt_ref,
               wcls_ref, bcls_ref, logits_ref, an_ref, ztopo_ref):
    dinv = dinv_ref[:, 0:1]
    s_tot = s_ref[0] + s_ref[1] + y_ref[...]
    h = jnp.maximum(dinv * s_tot + bg_ref[...], 0.0)
    z_topo = (
        jnp.dot(h, wpt_ref[...], preferred_element_type=jnp.float32)
        + bpt_ref[...]
    )
    logits_ref[...] = (
        jnp.dot(z_topo, wcls_ref[...], preferred_element_type=jnp.float32)
        + bcls_ref[...]
    )
    diff = z_topo - zsem_ref[...]
    an_ref[...] = jnp.sqrt(jnp.sum(diff * diff, axis=1))
    ztopo_ref[...] = z_topo


def _heads(s_parts, y, dinv, z_sem, b_gcn, W_pt, b_pt, W_cls, b_cls):
    return pl.pallas_call(
        _head_body,
        grid=(GRID,),
        in_specs=[
            pl.BlockSpec((2, BR, HID), lambda i: (0, i, 0)),
            pl.BlockSpec((BR, HID), lambda i: (i, 0)),
            pl.BlockSpec((BR, 16), lambda i: (i, 0)),
            pl.BlockSpec((BR, ALIGN), lambda i: (i, 0)),
            pl.BlockSpec((1, HID), lambda i: (0, 0)),
            pl.BlockSpec((HID, ALIGN), lambda i: (0, 0)),
            pl.BlockSpec((1, ALIGN), lambda i: (0, 0)),
            pl.BlockSpec((ALIGN, NUM_CLASSES), lambda i: (0, 0)),
            pl.BlockSpec((1, NUM_CLASSES), lambda i: (0, 0)),
        ],
        out_specs=[
            pl.BlockSpec((BR, NUM_CLASSES), lambda i: (i, 0)),
            pl.BlockSpec((BR,), lambda i: (i,)),
            pl.BlockSpec((BR, ALIGN), lambda i: (i, 0)),
        ],
        out_shape=[
            jax.ShapeDtypeStruct((N, NUM_CLASSES), jnp.float32),
            jax.ShapeDtypeStruct((N,), jnp.float32),
            jax.ShapeDtypeStruct((N, ALIGN), jnp.float32),
        ],
    )(s_parts, y, dinv, z_sem, b_gcn, W_pt, b_pt, W_cls, b_cls)


def kernel(x, edge_index, W_gcn, b_gcn, W_pt, b_pt, W_ps, b_ps, W_cls, b_cls):
    pad = E_PAD - E
    src_r = jnp.concatenate(
        [edge_index[0], jnp.zeros((pad,), jnp.int32)]).reshape(NC, NS, NCH, CH)
    dst_r = jnp.concatenate(
        [edge_index[1], jnp.full((pad,), DUMP, jnp.int32)]).reshape(
            NC, NS, NCH, CH)

    zeros16 = jnp.zeros((RPT, 16), jnp.float32)
    zeros64 = jnp.zeros((RPT, HID), jnp.float32)
    ones16 = jnp.ones((CH, 16), jnp.float32)

    deg_parts = _deg_kernel(dst_r, zeros16, ones16)
    xw, z_sem = _proj(x, W_gcn, W_ps, b_ps.reshape(1, ALIGN))
    y, dinv = _scale(deg_parts, xw)
    s_parts = _msg_kernel(src_r, dst_r, y, zeros64)
    logits, anomaly, z_topo = _heads(
        s_parts, y, dinv, z_sem, b_gcn.reshape(1, HID), W_pt,
        b_pt.reshape(1, ALIGN), W_cls, b_cls.reshape(1, NUM_CLASSES))
    return (logits, anomaly, z_topo, z_sem)


# speedup vs baseline: 41.2462x; 1.7332x over previous
import functools

import jax
import jax.numpy as jnp
from jax import lax
from jax.experimental import pallas as pl
from jax.experimental.pallas import tpu as pltpu
from jax.experimental.pallas import tpu_sc as plsc

N = 10000
E = 320000
IN_DIM = 128
HID = 64
ALIGN = 32
NUM_CLASSES = 7

NC = 2
NS = 16
CH = 128
NCH = 80
E_PAD = NC * NS * NCH * CH
DUMP = N
SROWS = 10240
RPT = SROWS // NS

BR = 2048
GRID = (N + BR - 1) // BR

_mesh = plsc.VectorSubcoreMesh(core_axis_name="c", subcore_axis_name="s")
_sc_params = pltpu.CompilerParams(use_tc_tiling_on_sc=False)


@functools.partial(
    pl.kernel,
    out_type=jax.ShapeDtypeStruct((NC, SROWS, 16), jnp.float32),
    mesh=_mesh,
    scratch_types=[
        pltpu.VMEM((NCH, CH), jnp.int32),
        pltpu.VMEM((CH, 16), jnp.float32),
        pltpu.VMEM_SHARED((SROWS, 16), jnp.float32),
    ],
    compiler_params=_sc_params,
)
def _deg_kernel(dst_hbm, zeros_hbm, ones_hbm, deg_out, dst_v, ones_v, acc_sh):
    cid = lax.axis_index("c")
    sid = lax.axis_index("s")
    pltpu.sync_copy(dst_hbm.at[cid, sid], dst_v)
    pltpu.sync_copy(ones_hbm, ones_v)

    pltpu.sync_copy(zeros_hbm, acc_sh.at[pl.ds(sid * RPT, RPT)])
    plsc.subcore_barrier()

    def chunk(j, _):
        pltpu.sync_copy(ones_v, acc_sh.at[dst_v.at[j]], add=True)
        return ()

    lax.fori_loop(0, NCH, chunk, ())
    plsc.subcore_barrier()
    pltpu.sync_copy(acc_sh.at[pl.ds(sid * RPT, RPT)],
                    deg_out.at[cid, pl.ds(sid * RPT, RPT)])


@functools.partial(
    pl.kernel,
    out_type=jax.ShapeDtypeStruct((NC, SROWS, HID), jnp.float32),
    mesh=_mesh,
    scratch_types=[
        pltpu.VMEM((NCH, CH), jnp.int32),
        pltpu.VMEM((NCH, CH), jnp.int32),
        pltpu.VMEM((CH, HID), jnp.float32),
        pltpu.VMEM((CH, HID), jnp.float32),
        pltpu.SemaphoreType.DMA,
        pltpu.SemaphoreType.DMA,
        pltpu.VMEM_SHARED((SROWS, HID), jnp.float32),
        pltpu.VMEM_SHARED((SROWS, HID), jnp.float32),
    ],
    compiler_params=_sc_params,
)
def _msg_kernel(src_hbm, dst_hbm, y_hbm, zeros_hbm, s_out,
                src_v, dst_v, buf0, buf1, sem0, sem1, acc_sh, y_sh):
    cid = lax.axis_index("c")
    sid = lax.axis_index("s")
    pltpu.sync_copy(src_hbm.at[cid, sid], src_v)
    pltpu.sync_copy(dst_hbm.at[cid, sid], dst_v)

    pltpu.sync_copy(y_hbm.at[pl.ds(sid * RPT, RPT)],
                    y_sh.at[pl.ds(sid * RPT, RPT)])
    pltpu.sync_copy(zeros_hbm, acc_sh.at[pl.ds(sid * RPT, RPT)])
    plsc.subcore_barrier()

    pltpu.async_copy(y_sh.at[src_v.at[0]], buf0, sem0)
    pltpu.async_copy(y_sh.at[src_v.at[1]], buf1, sem1)

    def pair(i, _):
        j0 = i * 2
        for b, (buf, sem) in enumerate(((buf0, sem0), (buf1, sem1))):
            j = j0 + b
            pltpu.make_async_copy(y_sh.at[src_v.at[j]], buf, sem).wait()
            pltpu.sync_copy(buf, acc_sh.at[dst_v.at[j]], add=True)

            @pl.when(j + 2 < NCH)
            def _():
                pltpu.async_copy(y_sh.at[src_v.at[j + 2]], buf, sem)

        return ()

    lax.fori_loop(0, NCH // 2, pair, ())
    plsc.subcore_barrier()
    pltpu.sync_copy(acc_sh.at[pl.ds(sid * RPT, RPT)],
                    s_out.at[cid, pl.ds(sid * RPT, RPT)])


def _proj_body(x_ref, wg_ref, wps_ref, bps_ref, xw_ref, zsem_ref):
    x = x_ref[...]
    xw_ref[...] = jnp.dot(x, wg_ref[...], preferred_element_type=jnp.float32)
    zsem_ref[...] = (
        jnp.dot(x, wps_ref[...], preferred_element_type=jnp.float32)
        + bps_ref[...]
    )


def _proj(x, W_gcn, W_ps, b_ps):
    return pl.pallas_call(
        _proj_body,
        grid=(GRID,),
        in_specs=[
            pl.BlockSpec((BR, IN_DIM), lambda i: (i, 0)),
            pl.BlockSpec((IN_DIM, HID), lambda i: (0, 0)),
            pl.BlockSpec((IN_DIM, ALIGN), lambda i: (0, 0)),
            pl.BlockSpec((1, ALIGN), lambda i: (0, 0)),
        ],
        out_specs=[
            pl.BlockSpec((BR, HID), lambda i: (i, 0)),
            pl.BlockSpec((BR, ALIGN), lambda i: (i, 0)),
        ],
        out_shape=[
            jax.ShapeDtypeStruct((N, HID), jnp.float32),
            jax.ShapeDtypeStruct((N, ALIGN), jnp.float32),
        ],
    )(x, W_gcn, W_ps, b_ps)


def _scale_body(dp_ref, xw_ref, y_ref, dinv_ref):
    deg = dp_ref[0, :, 0:1] + dp_ref[1, :, 0:1] + 1.0
    dinv = lax.rsqrt(deg)
    y_ref[...] = dinv * xw_ref[...]
    dinv_ref[...] = jnp.broadcast_to(dinv, dinv_ref.shape)


def _scale(deg_parts, xw):
    return pl.pallas_call(
        _scale_body,
        grid=(GRID,),
        in_specs=[
            pl.BlockSpec((2, BR, 16), lambda i: (0, i, 0)),
            pl.BlockSpec((BR, HID), lambda i: (i, 0)),
        ],
        out_specs=[
            pl.BlockSpec((BR, HID), lambda i: (i, 0)),
            pl.BlockSpec((BR, 16), lambda i: (i, 0)),
        ],
        out_shape=[
            jax.ShapeDtypeStruct((SROWS, HID), jnp.float32),
            jax.ShapeDtypeStruct((N, 16), jnp.float32),
        ],
    )(deg_parts, xw)


def _head_body(s_ref, y_ref, dinv_ref, zsem_ref, bg_ref, wpt_ref, bpt_ref,
               wcls_ref, bcls_ref, logits_ref, an_ref, ztopo_ref):
    dinv = dinv_ref[:, 0:1]
    s_tot = s_ref[0] + s_ref[1] + y_ref[...]
    h = jnp.maximum(dinv * s_tot + bg_ref[...], 0.0)
    z_topo = (
        jnp.dot(h, wpt_ref[...], preferred_element_type=jnp.float32)
        + bpt_ref[...]
    )
    logits_ref[...] = (
        jnp.dot(z_topo, wcls_ref[...], preferred_element_type=jnp.float32)
        + bcls_ref[...]
    )
    diff = z_topo - zsem_ref[...]
    an_ref[...] = jnp.sqrt(jnp.sum(diff * diff, axis=1))
    ztopo_ref[...] = z_topo


def _heads(s_parts, y, dinv, z_sem, b_gcn, W_pt, b_pt, W_cls, b_cls):
    return pl.pallas_call(
        _head_body,
        grid=(GRID,),
        in_specs=[
            pl.BlockSpec((2, BR, HID), lambda i: (0, i, 0)),
            pl.BlockSpec((BR, HID), lambda i: (i, 0)),
            pl.BlockSpec((BR, 16), lambda i: (i, 0)),
            pl.BlockSpec((BR, ALIGN), lambda i: (i, 0)),
            pl.BlockSpec((1, HID), lambda i: (0, 0)),
            pl.BlockSpec((HID, ALIGN), lambda i: (0, 0)),
            pl.BlockSpec((1, ALIGN), lambda i: (0, 0)),
            pl.BlockSpec((ALIGN, NUM_CLASSES), lambda i: (0, 0)),
            pl.BlockSpec((1, NUM_CLASSES), lambda i: (0, 0)),
        ],
        out_specs=[
            pl.BlockSpec((BR, NUM_CLASSES), lambda i: (i, 0)),
            pl.BlockSpec((BR,), lambda i: (i,)),
            pl.BlockSpec((BR, ALIGN), lambda i: (i, 0)),
        ],
        out_shape=[
            jax.ShapeDtypeStruct((N, NUM_CLASSES), jnp.float32),
            jax.ShapeDtypeStruct((N,), jnp.float32),
            jax.ShapeDtypeStruct((N, ALIGN), jnp.float32),
        ],
    )(s_parts, y, dinv, z_sem, b_gcn, W_pt, b_pt, W_cls, b_cls)


def kernel(x, edge_index, W_gcn, b_gcn, W_pt, b_pt, W_ps, b_ps, W_cls, b_cls):
    pad = E_PAD - E
    src_r = jnp.concatenate(
        [edge_index[0], jnp.zeros((pad,), jnp.int32)]).reshape(NC, NS, NCH, CH)
    dst_r = jnp.concatenate(
        [edge_index[1], jnp.full((pad,), DUMP, jnp.int32)]).reshape(
            NC, NS, NCH, CH)

    zeros16 = jnp.zeros((RPT, 16), jnp.float32)
    zeros64 = jnp.zeros((RPT, HID), jnp.float32)
    ones16 = jnp.ones((CH, 16), jnp.float32)

    deg_parts = _deg_kernel(dst_r, zeros16, ones16)
    xw, z_sem = _proj(x, W_gcn, W_ps, b_ps.reshape(1, ALIGN))
    y, dinv = _scale(deg_parts, xw)
    s_parts = _msg_kernel(src_r, dst_r, y, zeros64)
    logits, anomaly, z_topo = _heads(
        s_parts, y, dinv, z_sem, b_gcn.reshape(1, HID), W_pt,
        b_pt.reshape(1, ALIGN), W_cls, b_cls.reshape(1, NUM_CLASSES))
    return (logits, anomaly, z_topo, z_sem)
